# Initial kernel scaffold; baseline (speedup 1.0000x reference)
import jax
import jax.numpy as jnp
from jax.experimental import pallas as pl


def kernel(user, item, similar_explicit, similar_implicit, similar_explicit_ratings, user_bias, item_bias, item_q, item_x, item_y):
    raise NotImplementedError("write your pallas kernel here")


# speedup vs baseline: 5.2473x; 5.2473x over previous
import functools

import jax
import jax.numpy as jnp
from jax import lax
from jax.experimental import pallas as pl
from jax.experimental.pallas import tpu as pltpu
from jax.experimental.pallas import tpu_sc as plsc

B = 4096
L = 50
LP = 64
D = 64
AVG_RATING = 3.0
NC = 2
NS = 16
NW = NC * NS
CHUNK = B // NW
LG = 56


def _sc_body(user_r, item_r, sime_r, simi_r, rat_r, ub_t, ib_t, q_t, x_t, y_t,
             tbl_r, out_r, ubo_r, ibo_r,
             user_v, item_v, ub_v, ib_v, q_v, sime_v, simi_v, rat_v, tbl_v,
             xrows_v, yrows_v, bsim_v, w_v, wy_v, out_v, sem):
    wid = lax.axis_index("s") * NC + lax.axis_index("c")
    base = wid * CHUNK

    pltpu.sync_copy(user_r.at[pl.ds(base, CHUNK)], user_v)
    pltpu.sync_copy(item_r.at[pl.ds(base, CHUNK)], item_v)
    pltpu.sync_copy(sime_r.at[pl.ds(base, CHUNK)], sime_v)
    pltpu.sync_copy(simi_r.at[pl.ds(base, CHUNK)], simi_v)
    pltpu.sync_copy(rat_r.at[pl.ds(base, CHUNK)], rat_v)
    pltpu.sync_copy(tbl_r, tbl_v)

    pltpu.async_copy(ub_t.at[user_v], ub_v, sem).wait()
    pltpu.async_copy(ib_t.at[item_v], ib_v, sem).wait()
    pltpu.async_copy(q_t.at[item_v], q_v, sem).wait()
    pltpu.sync_copy(ub_v, ubo_r.at[pl.ds(base, CHUNK)])
    pltpu.sync_copy(ib_v, ibo_r.at[pl.ds(base, CHUNK)])

    def b_body(b, carry):
        cx_copy = pltpu.make_async_copy(x_t.at[sime_v.at[b, pl.ds(0, LG)]],
                                        xrows_v, sem)
        cy_copy = pltpu.make_async_copy(y_t.at[simi_v.at[b, pl.ds(0, LG)]],
                                        yrows_v, sem)
        cb_copy = pltpu.make_async_copy(ib_t.at[sime_v.at[b]], bsim_v, sem)
        cx_copy.start()
        cy_copy.start()
        cb_copy.start()
        cx_copy.wait()
        cy_copy.wait()
        cb_copy.wait()

        def bcast(ref, i):
            return plsc.load_gather(ref, [jnp.full((16,), i, jnp.int32)])

        ub_b = bcast(ub_v, b)
        mx = jnp.zeros((16,), jnp.float32)
        my = jnp.zeros((16,), jnp.float32)
        for c in range(LP // 16):
            sl = pl.ds(c * 16, 16)
            idx = sime_v[b, sl]
            m = idx > 0
            w = jnp.where(m, rat_v[b, sl] - (AVG_RATING + ub_b + bsim_v[sl]),
                          0.0)
            w_v[sl] = w
            mx = mx + jnp.where(m, 1.0, 0.0)
            idy = simi_v[b, sl]
            wy = jnp.where(idy > 0, 1.0, 0.0)
            wy_v[sl] = wy
            my = my + wy
        cx = bcast(tbl_v, jnp.sum(mx).astype(jnp.int32))
        cy = bcast(tbl_v, jnp.sum(my).astype(jnp.int32))

        accs = [jnp.zeros((16,), jnp.float32) for _ in range(8)]
        for n in range(L):
            w = bcast(w_v, n)
            wy = bcast(wy_v, n)
            for c in range(4):
                sl = pl.ds(c * 16, 16)
                accs[c] = accs[c] + w * xrows_v[n, sl]
                accs[4 + c] = accs[4 + c] + wy * yrows_v[n, sl]

        qacc = jnp.zeros((16,), jnp.float32)
        for c in range(4):
            sl = pl.ds(c * 16, 16)
            uf = cx * accs[c] + cy * accs[4 + c]
            qacc = qacc + uf * q_v[b, sl]
        res = AVG_RATING + ub_b + bcast(ib_v, b) + jnp.sum(qacc)
        lane0 = jnp.arange(16, dtype=jnp.int32) == 0
        plsc.store_scatter(out_v, [jnp.full((16,), b, jnp.int32)], res,
                           mask=lane0)
        return carry

    lax.fori_loop(0, CHUNK, b_body, 0)
    pltpu.sync_copy(out_v, out_r.at[pl.ds(base, CHUNK)])


def kernel(user, item, similar_explicit, similar_implicit,
           similar_explicit_ratings, user_bias, item_bias, item_q, item_x,
           item_y):
    pad = ((0, 0), (0, LP - L))
    sime_p = jnp.pad(similar_explicit.astype(jnp.int32), pad)
    simi_p = jnp.pad(similar_implicit.astype(jnp.int32), pad)
    rat_p = jnp.pad(similar_explicit_ratings, pad)
    tbl = 1.0 / (jnp.sqrt(jnp.arange(LP, dtype=jnp.float32)) + 1e-13)

    mesh = plsc.VectorSubcoreMesh(core_axis_name="c", subcore_axis_name="s")
    f32 = jnp.float32
    run = pl.kernel(
        _sc_body,
        mesh=mesh,
        compiler_params=pltpu.CompilerParams(needs_layout_passes=False,
                                              use_tc_tiling_on_sc=False),
        out_type=[jax.ShapeDtypeStruct((B,), f32) for _ in range(3)],
        scratch_types=[
            pltpu.VMEM((CHUNK,), jnp.int32),
            pltpu.VMEM((CHUNK,), jnp.int32),
            pltpu.VMEM((CHUNK,), f32),
            pltpu.VMEM((CHUNK,), f32),
            pltpu.VMEM((CHUNK, D), f32),
            pltpu.VMEM((CHUNK, LP), jnp.int32),
            pltpu.VMEM((CHUNK, LP), jnp.int32),
            pltpu.VMEM((CHUNK, LP), f32),
            pltpu.VMEM((LP,), f32),
            pltpu.VMEM((LG, D), f32),
            pltpu.VMEM((LG, D), f32),
            pltpu.VMEM((LP,), f32),
            pltpu.VMEM((LP,), f32),
            pltpu.VMEM((LP,), f32),
            pltpu.VMEM((CHUNK,), f32),
            pltpu.SemaphoreType.DMA,
        ],
    )
    out, ub, ib = run(user.astype(jnp.int32), item.astype(jnp.int32),
                      sime_p, simi_p, rat_p, user_bias, item_bias,
                      item_q, item_x, item_y, tbl)
    return (out, ub, ib)
